# baseline (speedup 1.0000x reference)
import functools

import jax
import jax.numpy as jnp
from jax import lax
from jax.experimental import pallas as pl
from jax.experimental.pallas import tpu as pltpu
from jax.experimental.pallas import tpu_sc as plsc

_N = 24256
_E = 388096
_D = 128
_B = 64
_R = 379
_RP = 384
_HF = 64
_NT = 16
_NC = 2
_BN = 3032

_f32 = jnp.float32
_i32 = jnp.int32


def _mesh():
    return plsc.VectorSubcoreMesh(core_axis_name="c", subcore_axis_name="s")


def _table_body(lew_ref, t_ref):
    a = lew_ref[...]
    t_ref[...] = 2.0 * jax.nn.sigmoid((a + a.T) * 0.5)


def _tc_table(lew_pad):
    return pl.pallas_call(
        _table_body,
        out_shape=jax.ShapeDtypeStruct((_RP, _RP), _f32),
    )(lew_pad)


def _deg_body(dp_ref, dinv_ref):
    dp = dp_ref[...]
    deg = dp[:8] + dp[8:] + 1.0
    dinv_ref[...] = lax.rsqrt(deg)


def _tc_deg(degp):
    return pl.pallas_call(
        _deg_body,
        out_shape=jax.ShapeDtypeStruct((8, _BN), _f32),
    )(degp)


def _mm_body(x_ref, w1_ref, we_ref, be_ref, dv_ref, hs1_ref, xe_ref):
    xb = x_ref[...]
    dv = dv_ref[...]
    hs1 = jnp.dot(xb, w1_ref[...], preferred_element_type=_f32) * dv
    xe = jnp.dot(xb, we_ref[...], preferred_element_type=_f32) + be_ref[...]
    xe = jnp.maximum(xe, 0.0)
    hs1_ref[0] = hs1[:, :_HF]
    hs1_ref[1] = hs1[:, _HF:]
    xe_ref[0] = xe[:, :_HF]
    xe_ref[1] = xe[:, _HF:]


def _tc_mm(x, w1, we, be_row, dvcol):
    grid = _N // _BN
    return pl.pallas_call(
        _mm_body,
        grid=(grid,),
        in_specs=[
            pl.BlockSpec((_BN, _D), lambda i: (i, 0)),
            pl.BlockSpec((_D, _D), lambda i: (0, 0)),
            pl.BlockSpec((_D, _D), lambda i: (0, 0)),
            pl.BlockSpec((1, _D), lambda i: (0, 0)),
            pl.BlockSpec((_BN, 1), lambda i: (i, 0)),
        ],
        out_specs=[
            pl.BlockSpec((2, _BN, _HF), lambda i: (0, i, 0)),
            pl.BlockSpec((2, _BN, _HF), lambda i: (0, i, 0)),
        ],
        out_shape=[
            jax.ShapeDtypeStruct((2, _N, _HF), _f32),
            jax.ShapeDtypeStruct((2, _N, _HF), _f32),
        ],
    )(x, w1, we, be_row, dvcol)


def _l1_body(agg_ref, hs1_ref, xe_ref, dv_ref, b1_ref, w2_ref, x1_ref, hs2_ref):
    dv = dv_ref[...]
    b1 = b1_ref[...]
    x1h = []
    for t in range(2):
        o = (agg_ref[t] + hs1_ref[t]) * dv + b1[:, _HF * t:_HF * (t + 1)]
        x1h.append(jnp.maximum(o, 0.0) + xe_ref[t])
    w2 = w2_ref[...]
    hs2 = (jnp.dot(x1h[0], w2[:_HF, :], preferred_element_type=_f32)
           + jnp.dot(x1h[1], w2[_HF:, :], preferred_element_type=_f32)) * dv
    x1_ref[0] = x1h[0]
    x1_ref[1] = x1h[1]
    hs2_ref[0] = hs2[:, :_HF]
    hs2_ref[1] = hs2[:, _HF:]


def _tc_l1(agg1, h1, xe, d2col, b1_row, w2):
    grid = _N // _BN
    half_spec = pl.BlockSpec((2, _BN, _HF), lambda i: (0, i, 0))
    return pl.pallas_call(
        _l1_body,
        grid=(grid,),
        in_specs=[
            half_spec,
            half_spec,
            half_spec,
            pl.BlockSpec((_BN, 1), lambda i: (i, 0)),
            pl.BlockSpec((1, _D), lambda i: (0, 0)),
            pl.BlockSpec((_D, _D), lambda i: (0, 0)),
        ],
        out_specs=[half_spec, half_spec],
        out_shape=[
            jax.ShapeDtypeStruct((2, _N, _HF), _f32),
            jax.ShapeDtypeStruct((2, _N, _HF), _f32),
        ],
    )(agg1, h1, xe, d2col, b1_row, w2)


def _l2_body(agg_ref, hs2_ref, x1_ref, dv_ref, b2_ref, x2_ref):
    dv = dv_ref[...]
    b2 = b2_ref[...]
    for t in range(2):
        o = (agg_ref[t] + hs2_ref[t]) * dv + b2[:, _HF * t:_HF * (t + 1)]
        x2_ref[t] = jnp.maximum(o, 0.0) + x1_ref[t]


def _tc_l2(agg2, h2, x1, d2col, b2_row):
    grid = _N // _BN
    half_spec = pl.BlockSpec((2, _BN, _HF), lambda i: (0, i, 0))
    return pl.pallas_call(
        _l2_body,
        grid=(grid,),
        in_specs=[
            half_spec,
            half_spec,
            half_spec,
            pl.BlockSpec((_BN, 1), lambda i: (i, 0)),
            pl.BlockSpec((1, _D), lambda i: (0, 0)),
        ],
        out_specs=half_spec,
        out_shape=jax.ShapeDtypeStruct((2, _N, _HF), _f32),
    )(agg2, h2, x1, d2col, b2_row)


def _pred_body(sums_ref, cnt_ref, wf_ref, bf_ref, out_ref):
    cnt = jnp.maximum(cnt_ref[...][:, 0:1], 1.0)
    wf = wf_ref[...]
    p0 = sums_ref[0] / cnt
    p1 = sums_ref[1] / cnt
    out_ref[...] = (jnp.dot(p0, wf[:_HF, :], preferred_element_type=_f32)
                    + jnp.dot(p1, wf[_HF:, :], preferred_element_type=_f32)
                    + bf_ref[...])


def _tc_pred(sums, cnt16, wf, bf_row):
    return pl.pallas_call(
        _pred_body,
        out_shape=jax.ShapeDtypeStruct((_B, 1), _f32),
    )(sums, cnt16, wf, bf_row)


_EW_TILE = _E // (_NC * _NT)
_EW_CH = 128
_EW_FULL = _EW_TILE // _EW_CH
_EW_TAIL = _EW_TILE - _EW_FULL * _EW_CH

_DEG_SL = _N // 8


_EW_PIPE = _EW_FULL - 1
_EW_K = (_EW_PIPE - 1) // 2


def _wdeg_body(row_h, col_h, tflat_h, w_h, degp_h,
               rbuf, cbuf, fbuf, wstage,
               rbuf2, cbuf2, fbuf2, wstage2,
               rtb, ctb, ftb, wtb,
               semi0, semi1, semt0, semt1, zb, degS):
    ci = lax.axis_index("c")
    s = lax.axis_index("s")
    wid = ci * _NT + s
    tbase = wid * _EW_TILE

    @pl.loop(0, 192)
    def _z(i):
        zb[pl.ds(i * 16, 16)] = jnp.zeros((16,), _f32)

    @pl.when(s < 8)
    def _zdeg():
        pltpu.sync_copy(zb.at[pl.ds(0, _DEG_SL)],
                        degS.at[pl.ds(s * _DEG_SL, _DEG_SL)])

    plsc.subcore_barrier()

    def _fcompute(ch, rb, cb, fb):
        @pl.loop(0, ch // 16)
        def _fg(g):
            sl = pl.ds(g * 16, 16)
            fb[sl] = (rb[sl] % _R) * _RP + (cb[sl] % _R)

    def _chunk_sync(base, ch, rb, cb, fb, wst):
        pltpu.sync_copy(row_h.at[pl.ds(base, ch)], rb)
        pltpu.sync_copy(col_h.at[pl.ds(base, ch)], cb)
        _fcompute(ch, rb, cb, fb)
        pltpu.sync_copy(tflat_h.at[fb], wst)
        pltpu.sync_copy(wst, w_h.at[pl.ds(base, ch)])
        pltpu.sync_copy(wst, degS.at[cb], add=True)

    _chunk_sync(tbase, _EW_TAIL, rtb, ctb, ftb, wtb)
    _chunk_sync(tbase + _EW_TAIL, _EW_CH, rbuf, cbuf, fbuf, wstage)

    pbase = tbase + _EW_TAIL + _EW_CH
    rb = (rbuf, rbuf2)
    cb = (cbuf, cbuf2)
    fb = (fbuf, fbuf2)
    wst = (wstage, wstage2)
    semi = (semi0, semi1)
    semt = (semt0, semt1)

    def start_idx(i, bs):
        base = pbase + i * _EW_CH
        pltpu.async_copy(row_h.at[pl.ds(base, _EW_CH)], rb[bs], semi[bs])
        pltpu.async_copy(col_h.at[pl.ds(base, _EW_CH)], cb[bs], semi[bs])

    def wait_idx(bs):
        pltpu.make_async_copy(row_h.at[pl.ds(tbase, _EW_CH)], rb[bs],
                              semi[bs]).wait()
        pltpu.make_async_copy(col_h.at[pl.ds(tbase, _EW_CH)], cb[bs],
                              semi[bs]).wait()

    def tg_start(bs):
        _fcompute(_EW_CH, rb[bs], cb[bs], fb[bs])
        pltpu.async_copy(tflat_h.at[fb[bs]], wst[bs], semt[bs])

    def tg_wait(bs):
        pltpu.make_async_copy(tflat_h.at[fb[bs]], wst[bs], semt[bs]).wait()

    def emit(i, bs):
        base = pbase + i * _EW_CH
        pltpu.sync_copy(wst[bs], w_h.at[pl.ds(base, _EW_CH)])
        pltpu.sync_copy(wst[bs], degS.at[cb[bs]], add=True)

    start_idx(0, 0)
    wait_idx(0)
    tg_start(0)
    start_idx(1, 1)

    @pl.loop(0, _EW_K)
    def _pair(k):
        i2 = k * 2
        wait_idx(1)
        tg_start(1)
        tg_wait(0)
        emit(i2, 0)
        start_idx(i2 + 2, 0)
        wait_idx(0)
        tg_start(0)
        tg_wait(1)
        emit(i2 + 1, 1)

        @pl.when(k < _EW_K - 1)
        def _pf():
            start_idx(i2 + 3, 1)

    tg_wait(0)
    emit(_EW_PIPE - 1, 0)

    plsc.subcore_barrier()

    @pl.when(s < 8)
    def _out():
        pltpu.sync_copy(degS.at[pl.ds(s * _DEG_SL, _DEG_SL)],
                        zb.at[pl.ds(0, _DEG_SL)])
        pltpu.sync_copy(zb.at[pl.ds(0, _DEG_SL)],
                        degp_h.at[pl.ds(ci * _N + s * _DEG_SL, _DEG_SL)])


def _sc_wdeg(row, col, tflat):
    k = pl.kernel(
        _wdeg_body,
        out_type=[
            jax.ShapeDtypeStruct((_E,), _f32),
            jax.ShapeDtypeStruct((2 * _N,), _f32),
        ],
        mesh=_mesh(),
        compiler_params=pltpu.CompilerParams(use_tc_tiling_on_sc=False),
        scratch_types=[
            pltpu.VMEM((_EW_CH,), _i32),
            pltpu.VMEM((_EW_CH,), _i32),
            pltpu.VMEM((_EW_CH,), _i32),
            pltpu.VMEM((_EW_CH,), _f32),
            pltpu.VMEM((_EW_CH,), _i32),
            pltpu.VMEM((_EW_CH,), _i32),
            pltpu.VMEM((_EW_CH,), _i32),
            pltpu.VMEM((_EW_CH,), _f32),
            pltpu.VMEM((_EW_TAIL,), _i32),
            pltpu.VMEM((_EW_TAIL,), _i32),
            pltpu.VMEM((_EW_TAIL,), _i32),
            pltpu.VMEM((_EW_TAIL,), _f32),
            pltpu.SemaphoreType.DMA,
            pltpu.SemaphoreType.DMA,
            pltpu.SemaphoreType.DMA,
            pltpu.SemaphoreType.DMA,
            pltpu.VMEM((3072,), _f32),
            pltpu.VMEM_SHARED((_N,), _f32),
        ],
    )
    return k(row, col, tflat)


_AG_TILE = _E // _NT
_AG_CH = 128
_AG_TAIL = 64
_AG_FULL = (_AG_TILE - _AG_TAIL) // _AG_CH
_AG_K = (_AG_FULL - 1) // 2
_ROWS_T = 1520
_ROWS_LAST = _N - 15 * _ROWS_T


def _agg_body(row_h, col_h, w_h, hf_h, agg_h,
              rbuf, cbuf, gbuf, wbuf, stage,
              rbuf2, cbuf2, gbuf2, wbuf2, stage2,
              rT, cT, gT, wT, stageT,
              semi0, semi1, semg0, semg1, zbuf, acc):
    ci = lax.axis_index("c")
    s = lax.axis_index("s")
    coff = ci * _N
    tbase = s * _AG_TILE

    @pl.loop(0, 128)
    def _z(i):
        for q in range(4):
            zbuf[i, pl.ds(q * 16, 16)] = jnp.zeros((16,), _f32)

    rbase = pl.multiple_of(s * _ROWS_T, 8)
    for kk in range(11):
        pltpu.sync_copy(zbuf, acc.at[pl.ds(rbase + kk * 128, 128)])

    @pl.when(s < _NT - 1)
    def _ztail():
        pltpu.sync_copy(zbuf.at[pl.ds(0, 112)],
                        acc.at[pl.ds(rbase + 1408, 112)])

    @pl.when(s == _NT - 1)
    def _ztail_last():
        pltpu.sync_copy(zbuf.at[pl.ds(0, 48)],
                        acc.at[pl.ds(rbase + 1408, 48)])

    plsc.subcore_barrier()

    rb = (rbuf, rbuf2)
    cb = (cbuf, cbuf2)
    gb = (gbuf, gbuf2)
    wb = (wbuf, wbuf2)
    st = (stage, stage2)
    semi = (semi0, semi1)
    semg = (semg0, semg1)
    fbase = tbase + _AG_TAIL

    def scale(stg, wref, n):
        @pl.loop(0, n // 16)
        def _sg(g4):
            off = g4 * 16
            wv = wref[pl.ds(off, 16)]
            for l in range(16):
                e = off + l
                nb = wv.at[lax.full((16,), l, _i32)].get(
                    mode="promise_in_bounds")
                for q in range(4):
                    sl2 = pl.ds(q * 16, 16)
                    stg[e, sl2] = stg[e, sl2] * nb

    pltpu.sync_copy(row_h.at[pl.ds(tbase, _AG_TAIL)], rT)
    pltpu.sync_copy(col_h.at[pl.ds(tbase, _AG_TAIL)], cT)
    pltpu.sync_copy(w_h.at[pl.ds(tbase, _AG_TAIL)], wT)
    for g in range(_AG_TAIL // 16):
        sl = pl.ds(g * 16, 16)
        gT[sl] = rT[sl] + coff
    pltpu.sync_copy(hf_h.at[gT], stageT)
    scale(stageT, wT, _AG_TAIL)
    pltpu.sync_copy(stageT, acc.at[cT], add=True)

    def start_idx(i, bs):
        base = fbase + i * _AG_CH
        pltpu.async_copy(row_h.at[pl.ds(base, _AG_CH)], rb[bs], semi[bs])
        pltpu.async_copy(col_h.at[pl.ds(base, _AG_CH)], cb[bs], semi[bs])
        pltpu.async_copy(w_h.at[pl.ds(base, _AG_CH)], wb[bs], semi[bs])

    def wait_idx(bs):
        pltpu.make_async_copy(row_h.at[pl.ds(tbase, _AG_CH)], rb[bs],
                              semi[bs]).wait()
        pltpu.make_async_copy(col_h.at[pl.ds(tbase, _AG_CH)], cb[bs],
                              semi[bs]).wait()
        pltpu.make_async_copy(w_h.at[pl.ds(tbase, _AG_CH)], wb[bs],
                              semi[bs]).wait()

    def gather_start(bs):
        for g in range(_AG_CH // 16):
            sl = pl.ds(g * 16, 16)
            gb[bs][sl] = rb[bs][sl] + coff
        pltpu.async_copy(hf_h.at[gb[bs]], st[bs], semg[bs])

    def gather_wait(bs):
        pltpu.make_async_copy(hf_h.at[gb[bs]], st[bs], semg[bs]).wait()

    def scale_scatter(bs):
        scale(st[bs], wb[bs], _AG_CH)
        pltpu.sync_copy(st[bs], acc.at[cb[bs]], add=True)

    start_idx(0, 0)
    wait_idx(0)
    gather_start(0)
    start_idx(1, 1)

    @pl.loop(0, _AG_K)
    def _pair(k):
        i2 = k * 2
        wait_idx(1)
        gather_start(1)
        gather_wait(0)
        scale_scatter(0)
        start_idx(i2 + 2, 0)
        wait_idx(0)
        gather_start(0)
        gather_wait(1)
        scale_scatter(1)

        @pl.when(k < _AG_K - 1)
        def _pf():
            start_idx(i2 + 3, 1)

    gather_wait(0)
    scale_scatter(0)

    plsc.subcore_barrier()

    obase = pl.multiple_of(coff + rbase, 8)
    for kk in range(11):
        pltpu.sync_copy(acc.at[pl.ds(rbase + kk * 128, 128)], zbuf)
        pltpu.sync_copy(zbuf, agg_h.at[pl.ds(obase + kk * 128, 128)])

    @pl.when(s < _NT - 1)
    def _otail():
        pltpu.sync_copy(acc.at[pl.ds(rbase + 1408, 112)],
                        zbuf.at[pl.ds(0, 112)])
        pltpu.sync_copy(zbuf.at[pl.ds(0, 112)],
                        agg_h.at[pl.ds(obase + 1408, 112)])

    @pl.when(s == _NT - 1)
    def _otail_last():
        pltpu.sync_copy(acc.at[pl.ds(rbase + 1408, 48)],
                        zbuf.at[pl.ds(0, 48)])
        pltpu.sync_copy(zbuf.at[pl.ds(0, 48)],
                        agg_h.at[pl.ds(obase + 1408, 48)])


def _sc_agg(row, col, w, hflat):
    k = pl.kernel(
        _agg_body,
        out_type=jax.ShapeDtypeStruct((2 * _N, _HF), _f32),
        mesh=_mesh(),
        compiler_params=pltpu.CompilerParams(use_tc_tiling_on_sc=False),
        scratch_types=[
            pltpu.VMEM((_AG_CH,), _i32),
            pltpu.VMEM((_AG_CH,), _i32),
            pltpu.VMEM((_AG_CH,), _i32),
            pltpu.VMEM((_AG_CH,), _f32),
            pltpu.VMEM((_AG_CH, _HF), _f32),
            pltpu.VMEM((_AG_CH,), _i32),
            pltpu.VMEM((_AG_CH,), _i32),
            pltpu.VMEM((_AG_CH,), _i32),
            pltpu.VMEM((_AG_CH,), _f32),
            pltpu.VMEM((_AG_CH, _HF), _f32),
            pltpu.VMEM((_AG_TAIL,), _i32),
            pltpu.VMEM((_AG_TAIL,), _i32),
            pltpu.VMEM((_AG_TAIL,), _i32),
            pltpu.VMEM((_AG_TAIL,), _f32),
            pltpu.VMEM((_AG_TAIL, _HF), _f32),
            pltpu.SemaphoreType.DMA,
            pltpu.SemaphoreType.DMA,
            pltpu.SemaphoreType.DMA,
            pltpu.SemaphoreType.DMA,
            pltpu.VMEM((128, _HF), _f32),
            pltpu.VMEM_SHARED((_N, _HF), _f32),
        ],
    )
    return k(row, col, w, hflat)


_PL_FULL = _N // 128
_PL_TAIL = _N - _PL_FULL * 128


def _pool_body(xf_h, batch_h, sums_h, cnt_h,
               bbuf, btail, stage, onesv, zbv, zcv, sumS, cntS):
    ci = lax.axis_index("c")
    s = lax.axis_index("s")
    coff = ci * _N

    @pl.loop(0, 64)
    def _z(i):
        for q in range(4):
            zbv[i, pl.ds(q * 16, 16)] = jnp.zeros((16,), _f32)

    @pl.loop(0, 128)
    def _o(i):
        onesv[i, pl.ds(0, 16)] = jnp.ones((16,), _f32)

    @pl.loop(0, 64)
    def _zc(i):
        zcv[i, pl.ds(0, 16)] = jnp.zeros((16,), _f32)

    @pl.when(s == 0)
    def _zero():
        pltpu.sync_copy(zbv, sumS)

    @pl.when(jnp.logical_and(s == 0, ci == 0))
    def _zeroc():
        pltpu.sync_copy(zcv, cntS)

    plsc.subcore_barrier()

    nch = (204 - s) // 16

    @pl.loop(0, nch)
    def _c(k):
        j = s + k * _NT
        base = j * 128
        pltpu.sync_copy(batch_h.at[pl.ds(base, 128)], bbuf)
        pltpu.sync_copy(xf_h.at[pl.ds(coff + base, 128)], stage)
        pltpu.sync_copy(stage, sumS.at[bbuf], add=True)

        @pl.when(ci == 0)
        def _cnt():
            pltpu.sync_copy(onesv, cntS.at[bbuf], add=True)

    @pl.when(s == _NT - 1)
    def _tail():
        base = _PL_FULL * 128
        pltpu.sync_copy(batch_h.at[pl.ds(base, _PL_TAIL)], btail)
        pltpu.sync_copy(xf_h.at[pl.ds(coff + base, _PL_TAIL)],
                        stage.at[pl.ds(0, _PL_TAIL)])
        pltpu.sync_copy(stage.at[pl.ds(0, _PL_TAIL)],
                        sumS.at[btail], add=True)

        @pl.when(ci == 0)
        def _cntt():
            pltpu.sync_copy(onesv.at[pl.ds(0, _PL_TAIL)],
                            cntS.at[btail], add=True)

    plsc.subcore_barrier()

    @pl.when(s == 0)
    def _out():
        pltpu.sync_copy(sumS, zbv)
        pltpu.sync_copy(zbv, sums_h.at[ci])

    @pl.when(jnp.logical_and(s == 0, ci == 0))
    def _outc():
        pltpu.sync_copy(cntS, zcv)
        pltpu.sync_copy(zcv, cnt_h)


def _sc_pool(xflat, batch):
    k = pl.kernel(
        _pool_body,
        out_type=[
            jax.ShapeDtypeStruct((2, _B, _HF), _f32),
            jax.ShapeDtypeStruct((_B, 16), _f32),
        ],
        mesh=_mesh(),
        compiler_params=pltpu.CompilerParams(use_tc_tiling_on_sc=False),
        scratch_types=[
            pltpu.VMEM((128,), _i32),
            pltpu.VMEM((_PL_TAIL,), _i32),
            pltpu.VMEM((128, _HF), _f32),
            pltpu.VMEM((128, 16), _f32),
            pltpu.VMEM((_B, _HF), _f32),
            pltpu.VMEM((_B, 16), _f32),
            pltpu.VMEM_SHARED((_B, _HF), _f32),
            pltpu.VMEM_SHARED((_B, 16), _f32),
        ],
    )
    return k(xflat, batch)


def kernel(x, edge_index, edge_weight, batch, W1, b1, W2, b2, We, be, Wf, bf, lew):
    del edge_weight
    row = edge_index[0]
    col = edge_index[1]

    lew_pad = jnp.pad(lew, ((0, _RP - _R), (0, _RP - _R)))
    tflat = _tc_table(lew_pad).reshape(-1)

    w, degpf = _sc_wdeg(row, col, tflat)
    dinv8 = _tc_deg(degpf.reshape(16, _BN))
    dvcol = dinv8.reshape(-1, 1)

    hs1, xe = _tc_mm(x, W1, We, be.reshape(1, _D), dvcol)
    agg1 = _sc_agg(row, col, w, hs1.reshape(2 * _N, _HF))
    x1, hs2 = _tc_l1(agg1.reshape(2, _N, _HF), hs1, xe, dvcol,
                     b1.reshape(1, _D), W2)
    agg2 = _sc_agg(row, col, w, hs2.reshape(2 * _N, _HF))
    x2 = _tc_l2(agg2.reshape(2, _N, _HF), hs2, x1, dvcol, b2.reshape(1, _D))

    sums, cnt16 = _sc_pool(x2.reshape(2 * _N, _HF), batch)
    return _tc_pred(sums, cnt16, Wf, bf.reshape(1, 1))


# speedup vs baseline: 1.7209x; 1.7209x over previous
import functools

import jax
import jax.numpy as jnp
from jax import lax
from jax.experimental import pallas as pl
from jax.experimental.pallas import tpu as pltpu
from jax.experimental.pallas import tpu_sc as plsc

_N = 24256
_E = 388096
_D = 128
_B = 64
_R = 379
_RP = 384
_HF = 64
_NT = 16
_NC = 2
_BN = 3032

_f32 = jnp.float32
_i32 = jnp.int32


def _mesh():
    return plsc.VectorSubcoreMesh(core_axis_name="c", subcore_axis_name="s")


def _table_body(lew_ref, t_ref):
    a = lew_ref[...]
    t_ref[...] = 2.0 * jax.nn.sigmoid((a + a.T) * 0.5)


def _tc_table(lew_pad):
    return pl.pallas_call(
        _table_body,
        out_shape=jax.ShapeDtypeStruct((_RP, _RP), _f32),
    )(lew_pad)


def _deg_body(dp_ref, dinv_ref):
    dp = dp_ref[...]
    deg = dp[:8] + dp[8:] + 1.0
    dinv_ref[...] = lax.rsqrt(deg)


def _tc_deg(degp):
    return pl.pallas_call(
        _deg_body,
        out_shape=jax.ShapeDtypeStruct((8, _BN), _f32),
    )(degp)


def _mm_body(x_ref, w1_ref, we_ref, be_ref, dv_ref, hs1_ref, xe_ref):
    xb = x_ref[...]
    dv = dv_ref[...]
    hs1 = jnp.dot(xb, w1_ref[...], preferred_element_type=_f32) * dv
    xe = jnp.dot(xb, we_ref[...], preferred_element_type=_f32) + be_ref[...]
    xe = jnp.maximum(xe, 0.0)
    hs1_ref[0] = hs1[:, :_HF]
    hs1_ref[1] = hs1[:, _HF:]
    xe_ref[0] = xe[:, :_HF]
    xe_ref[1] = xe[:, _HF:]


def _tc_mm(x, w1, we, be_row, dvcol):
    grid = _N // _BN
    return pl.pallas_call(
        _mm_body,
        grid=(grid,),
        in_specs=[
            pl.BlockSpec((_BN, _D), lambda i: (i, 0)),
            pl.BlockSpec((_D, _D), lambda i: (0, 0)),
            pl.BlockSpec((_D, _D), lambda i: (0, 0)),
            pl.BlockSpec((1, _D), lambda i: (0, 0)),
            pl.BlockSpec((_BN, 1), lambda i: (i, 0)),
        ],
        out_specs=[
            pl.BlockSpec((2, _BN, _HF), lambda i: (0, i, 0)),
            pl.BlockSpec((2, _BN, _HF), lambda i: (0, i, 0)),
        ],
        out_shape=[
            jax.ShapeDtypeStruct((2, _N, _HF), _f32),
            jax.ShapeDtypeStruct((2, _N, _HF), _f32),
        ],
    )(x, w1, we, be_row, dvcol)


def _l1_body(agg_ref, hs1_ref, xe_ref, dv_ref, b1_ref, w2_ref, x1_ref, hs2_ref):
    dv = dv_ref[...]
    b1 = b1_ref[...]
    x1h = []
    for t in range(2):
        o = (agg_ref[t] + hs1_ref[t]) * dv + b1[:, _HF * t:_HF * (t + 1)]
        x1h.append(jnp.maximum(o, 0.0) + xe_ref[t])
    w2 = w2_ref[...]
    hs2 = (jnp.dot(x1h[0], w2[:_HF, :], preferred_element_type=_f32)
           + jnp.dot(x1h[1], w2[_HF:, :], preferred_element_type=_f32)) * dv
    x1_ref[0] = x1h[0]
    x1_ref[1] = x1h[1]
    hs2_ref[0] = hs2[:, :_HF]
    hs2_ref[1] = hs2[:, _HF:]


def _tc_l1(agg1, h1, xe, d2col, b1_row, w2):
    grid = _N // _BN
    half_spec = pl.BlockSpec((2, _BN, _HF), lambda i: (0, i, 0))
    return pl.pallas_call(
        _l1_body,
        grid=(grid,),
        in_specs=[
            half_spec,
            half_spec,
            half_spec,
            pl.BlockSpec((_BN, 1), lambda i: (i, 0)),
            pl.BlockSpec((1, _D), lambda i: (0, 0)),
            pl.BlockSpec((_D, _D), lambda i: (0, 0)),
        ],
        out_specs=[half_spec, half_spec],
        out_shape=[
            jax.ShapeDtypeStruct((2, _N, _HF), _f32),
            jax.ShapeDtypeStruct((2, _N, _HF), _f32),
        ],
    )(agg1, h1, xe, d2col, b1_row, w2)


def _l2_body(agg_ref, hs2_ref, x1_ref, dv_ref, b2_ref, x2_ref):
    dv = dv_ref[...]
    b2 = b2_ref[...]
    for t in range(2):
        o = (agg_ref[t] + hs2_ref[t]) * dv + b2[:, _HF * t:_HF * (t + 1)]
        x2_ref[t] = jnp.maximum(o, 0.0) + x1_ref[t]


def _tc_l2(agg2, h2, x1, d2col, b2_row):
    grid = _N // _BN
    half_spec = pl.BlockSpec((2, _BN, _HF), lambda i: (0, i, 0))
    return pl.pallas_call(
        _l2_body,
        grid=(grid,),
        in_specs=[
            half_spec,
            half_spec,
            half_spec,
            pl.BlockSpec((_BN, 1), lambda i: (i, 0)),
            pl.BlockSpec((1, _D), lambda i: (0, 0)),
        ],
        out_specs=half_spec,
        out_shape=jax.ShapeDtypeStruct((2, _N, _HF), _f32),
    )(agg2, h2, x1, d2col, b2_row)


def _pred_body(sums_ref, cnt_ref, wf_ref, bf_ref, out_ref):
    cnt = jnp.maximum(cnt_ref[...][:, 0:1], 1.0)
    wf = wf_ref[...]
    p0 = sums_ref[0] / cnt
    p1 = sums_ref[1] / cnt
    out_ref[...] = (jnp.dot(p0, wf[:_HF, :], preferred_element_type=_f32)
                    + jnp.dot(p1, wf[_HF:, :], preferred_element_type=_f32)
                    + bf_ref[...])


def _tc_pred(sums, cnt16, wf, bf_row):
    return pl.pallas_call(
        _pred_body,
        out_shape=jax.ShapeDtypeStruct((_B, 1), _f32),
    )(sums, cnt16, wf, bf_row)


_EW_TILE = _E // (_NC * _NT)
_EW_CH = 128
_EW_FULL = _EW_TILE // _EW_CH
_EW_TAIL = _EW_TILE - _EW_FULL * _EW_CH

_DEG_SL = _N // 8


_EW_PIPE = _EW_FULL - 1
_EW_K = (_EW_PIPE - 1) // 2


def _wdeg_body(row_h, col_h, tflat_h, w_h, degp_h,
               rbuf, cbuf, fbuf, wstage,
               rbuf2, cbuf2, fbuf2, wstage2,
               rtb, ctb, ftb, wtb,
               semi0, semi1, semt0, semt1, zb, degS):
    ci = lax.axis_index("c")
    s = lax.axis_index("s")
    wid = ci * _NT + s
    tbase = wid * _EW_TILE

    @pl.loop(0, 192)
    def _z(i):
        zb[pl.ds(i * 16, 16)] = jnp.zeros((16,), _f32)

    @pl.when(s < 8)
    def _zdeg():
        pltpu.sync_copy(zb.at[pl.ds(0, _DEG_SL)],
                        degS.at[pl.ds(s * _DEG_SL, _DEG_SL)])

    plsc.subcore_barrier()

    def _fcompute(ch, rb, cb, fb):
        @pl.loop(0, ch // 16)
        def _fg(g):
            sl = pl.ds(g * 16, 16)
            fb[sl] = (rb[sl] % _R) * _RP + (cb[sl] % _R)

    def _chunk_sync(base, ch, rb, cb, fb, wst):
        pltpu.sync_copy(row_h.at[pl.ds(base, ch)], rb)
        pltpu.sync_copy(col_h.at[pl.ds(base, ch)], cb)
        _fcompute(ch, rb, cb, fb)
        pltpu.sync_copy(tflat_h.at[fb], wst)
        pltpu.sync_copy(wst, w_h.at[pl.ds(base, ch)])
        pltpu.sync_copy(wst, degS.at[cb], add=True)

    _chunk_sync(tbase, _EW_TAIL, rtb, ctb, ftb, wtb)
    _chunk_sync(tbase + _EW_TAIL, _EW_CH, rbuf, cbuf, fbuf, wstage)

    pbase = tbase + _EW_TAIL + _EW_CH
    rb = (rbuf, rbuf2)
    cb = (cbuf, cbuf2)
    fb = (fbuf, fbuf2)
    wst = (wstage, wstage2)
    semi = (semi0, semi1)
    semt = (semt0, semt1)

    def start_idx(i, bs):
        base = pbase + i * _EW_CH
        pltpu.async_copy(row_h.at[pl.ds(base, _EW_CH)], rb[bs], semi[bs])
        pltpu.async_copy(col_h.at[pl.ds(base, _EW_CH)], cb[bs], semi[bs])

    def wait_idx(bs):
        pltpu.make_async_copy(row_h.at[pl.ds(tbase, _EW_CH)], rb[bs],
                              semi[bs]).wait()
        pltpu.make_async_copy(col_h.at[pl.ds(tbase, _EW_CH)], cb[bs],
                              semi[bs]).wait()

    def tg_start(bs):
        _fcompute(_EW_CH, rb[bs], cb[bs], fb[bs])
        pltpu.async_copy(tflat_h.at[fb[bs]], wst[bs], semt[bs])

    def tg_wait(bs):
        pltpu.make_async_copy(tflat_h.at[fb[bs]], wst[bs], semt[bs]).wait()

    def emit(i, bs):
        base = pbase + i * _EW_CH
        pltpu.sync_copy(wst[bs], w_h.at[pl.ds(base, _EW_CH)])
        pltpu.sync_copy(wst[bs], degS.at[cb[bs]], add=True)

    start_idx(0, 0)
    wait_idx(0)
    tg_start(0)
    start_idx(1, 1)

    @pl.loop(0, _EW_K)
    def _pair(k):
        i2 = k * 2
        wait_idx(1)
        tg_start(1)
        tg_wait(0)
        emit(i2, 0)
        start_idx(i2 + 2, 0)
        wait_idx(0)
        tg_start(0)
        tg_wait(1)
        emit(i2 + 1, 1)

        @pl.when(k < _EW_K - 1)
        def _pf():
            start_idx(i2 + 3, 1)

    tg_wait(0)
    emit(_EW_PIPE - 1, 0)

    plsc.subcore_barrier()

    @pl.when(s < 8)
    def _out():
        pltpu.sync_copy(degS.at[pl.ds(s * _DEG_SL, _DEG_SL)],
                        zb.at[pl.ds(0, _DEG_SL)])
        pltpu.sync_copy(zb.at[pl.ds(0, _DEG_SL)],
                        degp_h.at[pl.ds(ci * _N + s * _DEG_SL, _DEG_SL)])


def _sc_wdeg(row, col, tflat):
    k = pl.kernel(
        _wdeg_body,
        out_type=[
            jax.ShapeDtypeStruct((_E,), _f32),
            jax.ShapeDtypeStruct((2 * _N,), _f32),
        ],
        mesh=_mesh(),
        compiler_params=pltpu.CompilerParams(use_tc_tiling_on_sc=False),
        scratch_types=[
            pltpu.VMEM((_EW_CH,), _i32),
            pltpu.VMEM((_EW_CH,), _i32),
            pltpu.VMEM((_EW_CH,), _i32),
            pltpu.VMEM((_EW_CH,), _f32),
            pltpu.VMEM((_EW_CH,), _i32),
            pltpu.VMEM((_EW_CH,), _i32),
            pltpu.VMEM((_EW_CH,), _i32),
            pltpu.VMEM((_EW_CH,), _f32),
            pltpu.VMEM((_EW_TAIL,), _i32),
            pltpu.VMEM((_EW_TAIL,), _i32),
            pltpu.VMEM((_EW_TAIL,), _i32),
            pltpu.VMEM((_EW_TAIL,), _f32),
            pltpu.SemaphoreType.DMA,
            pltpu.SemaphoreType.DMA,
            pltpu.SemaphoreType.DMA,
            pltpu.SemaphoreType.DMA,
            pltpu.VMEM((3072,), _f32),
            pltpu.VMEM_SHARED((_N,), _f32),
        ],
    )
    return k(row, col, tflat)


_AG_TILE = _E // _NT
_AG_CH = 128
_AG_TAIL = 64
_AG_FULL = (_AG_TILE - _AG_TAIL) // _AG_CH
_AG_K = (_AG_FULL - 1) // 2
_ROWS_T = 1520
_ROWS_LAST = _N - 15 * _ROWS_T


def _agg_body(row_h, col_h, w_h, hf_h, agg_h,
              rbuf, cbuf, gbuf, wbuf, stage,
              rbuf2, cbuf2, gbuf2, wbuf2, stage2,
              rT, cT, gT, wT, stageT,
              semi0, semi1, semg0, semg1, zbuf, acc):
    ci = lax.axis_index("c")
    s = lax.axis_index("s")
    coff = ci * _N
    tbase = s * _AG_TILE

    @pl.loop(0, 128)
    def _z(i):
        for q in range(4):
            zbuf[i, pl.ds(q * 16, 16)] = jnp.zeros((16,), _f32)

    rbase = pl.multiple_of(s * _ROWS_T, 8)
    for kk in range(11):
        pltpu.sync_copy(zbuf, acc.at[pl.ds(rbase + kk * 128, 128)])

    @pl.when(s < _NT - 1)
    def _ztail():
        pltpu.sync_copy(zbuf.at[pl.ds(0, 112)],
                        acc.at[pl.ds(rbase + 1408, 112)])

    @pl.when(s == _NT - 1)
    def _ztail_last():
        pltpu.sync_copy(zbuf.at[pl.ds(0, 48)],
                        acc.at[pl.ds(rbase + 1408, 48)])

    plsc.subcore_barrier()

    rb = (rbuf, rbuf2)
    cb = (cbuf, cbuf2)
    gb = (gbuf, gbuf2)
    wb = (wbuf, wbuf2)
    st = (stage, stage2)
    semi = (semi0, semi1)
    semg = (semg0, semg1)
    fbase = tbase + _AG_TAIL

    def scale(stg, wref, n):
        for g4 in range(n // 16):
            wv = wref[pl.ds(g4 * 16, 16)]
            for l in range(16):
                e = g4 * 16 + l
                nb = wv.at[lax.full((16,), l, _i32)].get(
                    mode="promise_in_bounds")
                for q in range(4):
                    sl2 = pl.ds(q * 16, 16)
                    stg[e, sl2] = stg[e, sl2] * nb

    pltpu.sync_copy(row_h.at[pl.ds(tbase, _AG_TAIL)], rT)
    pltpu.sync_copy(col_h.at[pl.ds(tbase, _AG_TAIL)], cT)
    pltpu.sync_copy(w_h.at[pl.ds(tbase, _AG_TAIL)], wT)
    for g in range(_AG_TAIL // 16):
        sl = pl.ds(g * 16, 16)
        gT[sl] = rT[sl] + coff
    pltpu.sync_copy(hf_h.at[gT], stageT)
    scale(stageT, wT, _AG_TAIL)
    pltpu.sync_copy(stageT, acc.at[cT], add=True)

    def start_idx(i, bs):
        base = fbase + i * _AG_CH
        pltpu.async_copy(row_h.at[pl.ds(base, _AG_CH)], rb[bs], semi[bs])
        pltpu.async_copy(col_h.at[pl.ds(base, _AG_CH)], cb[bs], semi[bs])
        pltpu.async_copy(w_h.at[pl.ds(base, _AG_CH)], wb[bs], semi[bs])

    def wait_idx(bs):
        pltpu.make_async_copy(row_h.at[pl.ds(tbase, _AG_CH)], rb[bs],
                              semi[bs]).wait()
        pltpu.make_async_copy(col_h.at[pl.ds(tbase, _AG_CH)], cb[bs],
                              semi[bs]).wait()
        pltpu.make_async_copy(w_h.at[pl.ds(tbase, _AG_CH)], wb[bs],
                              semi[bs]).wait()

    def gather_start(bs):
        for g in range(_AG_CH // 16):
            sl = pl.ds(g * 16, 16)
            gb[bs][sl] = rb[bs][sl] + coff
        pltpu.async_copy(hf_h.at[gb[bs]], st[bs], semg[bs])

    def gather_wait(bs):
        pltpu.make_async_copy(hf_h.at[gb[bs]], st[bs], semg[bs]).wait()

    def scale_scatter(bs):
        scale(st[bs], wb[bs], _AG_CH)
        pltpu.sync_copy(st[bs], acc.at[cb[bs]], add=True)

    start_idx(0, 0)
    wait_idx(0)
    gather_start(0)
    start_idx(1, 1)

    @pl.loop(0, _AG_K)
    def _pair(k):
        i2 = k * 2
        wait_idx(1)
        gather_start(1)
        gather_wait(0)
        scale_scatter(0)
        start_idx(i2 + 2, 0)
        wait_idx(0)
        gather_start(0)
        gather_wait(1)
        scale_scatter(1)

        @pl.when(k < _AG_K - 1)
        def _pf():
            start_idx(i2 + 3, 1)

    gather_wait(0)
    scale_scatter(0)

    plsc.subcore_barrier()

    obase = pl.multiple_of(coff + rbase, 8)
    for kk in range(11):
        pltpu.sync_copy(acc.at[pl.ds(rbase + kk * 128, 128)], zbuf)
        pltpu.sync_copy(zbuf, agg_h.at[pl.ds(obase + kk * 128, 128)])

    @pl.when(s < _NT - 1)
    def _otail():
        pltpu.sync_copy(acc.at[pl.ds(rbase + 1408, 112)],
                        zbuf.at[pl.ds(0, 112)])
        pltpu.sync_copy(zbuf.at[pl.ds(0, 112)],
                        agg_h.at[pl.ds(obase + 1408, 112)])

    @pl.when(s == _NT - 1)
    def _otail_last():
        pltpu.sync_copy(acc.at[pl.ds(rbase + 1408, 48)],
                        zbuf.at[pl.ds(0, 48)])
        pltpu.sync_copy(zbuf.at[pl.ds(0, 48)],
                        agg_h.at[pl.ds(obase + 1408, 48)])


def _sc_agg(row, col, w, hflat):
    k = pl.kernel(
        _agg_body,
        out_type=jax.ShapeDtypeStruct((2 * _N, _HF), _f32),
        mesh=_mesh(),
        compiler_params=pltpu.CompilerParams(use_tc_tiling_on_sc=False),
        scratch_types=[
            pltpu.VMEM((_AG_CH,), _i32),
            pltpu.VMEM((_AG_CH,), _i32),
            pltpu.VMEM((_AG_CH,), _i32),
            pltpu.VMEM((_AG_CH,), _f32),
            pltpu.VMEM((_AG_CH, _HF), _f32),
            pltpu.VMEM((_AG_CH,), _i32),
            pltpu.VMEM((_AG_CH,), _i32),
            pltpu.VMEM((_AG_CH,), _i32),
            pltpu.VMEM((_AG_CH,), _f32),
            pltpu.VMEM((_AG_CH, _HF), _f32),
            pltpu.VMEM((_AG_TAIL,), _i32),
            pltpu.VMEM((_AG_TAIL,), _i32),
            pltpu.VMEM((_AG_TAIL,), _i32),
            pltpu.VMEM((_AG_TAIL,), _f32),
            pltpu.VMEM((_AG_TAIL, _HF), _f32),
            pltpu.SemaphoreType.DMA,
            pltpu.SemaphoreType.DMA,
            pltpu.SemaphoreType.DMA,
            pltpu.SemaphoreType.DMA,
            pltpu.VMEM((128, _HF), _f32),
            pltpu.VMEM_SHARED((_N, _HF), _f32),
        ],
    )
    return k(row, col, w, hflat)


_PL_FULL = _N // 128
_PL_TAIL = _N - _PL_FULL * 128


def _pool_body(xf_h, batch_h, sums_h, cnt_h,
               bbuf, btail, stage, onesv, zbv, zcv, sumS, cntS):
    ci = lax.axis_index("c")
    s = lax.axis_index("s")
    coff = ci * _N

    @pl.loop(0, 64)
    def _z(i):
        for q in range(4):
            zbv[i, pl.ds(q * 16, 16)] = jnp.zeros((16,), _f32)

    @pl.loop(0, 128)
    def _o(i):
        onesv[i, pl.ds(0, 16)] = jnp.ones((16,), _f32)

    @pl.loop(0, 64)
    def _zc(i):
        zcv[i, pl.ds(0, 16)] = jnp.zeros((16,), _f32)

    @pl.when(s == 0)
    def _zero():
        pltpu.sync_copy(zbv, sumS)

    @pl.when(jnp.logical_and(s == 0, ci == 0))
    def _zeroc():
        pltpu.sync_copy(zcv, cntS)

    plsc.subcore_barrier()

    nch = (204 - s) // 16

    @pl.loop(0, nch)
    def _c(k):
        j = s + k * _NT
        base = j * 128
        pltpu.sync_copy(batch_h.at[pl.ds(base, 128)], bbuf)
        pltpu.sync_copy(xf_h.at[pl.ds(coff + base, 128)], stage)
        pltpu.sync_copy(stage, sumS.at[bbuf], add=True)

        @pl.when(ci == 0)
        def _cnt():
            pltpu.sync_copy(onesv, cntS.at[bbuf], add=True)

    @pl.when(s == _NT - 1)
    def _tail():
        base = _PL_FULL * 128
        pltpu.sync_copy(batch_h.at[pl.ds(base, _PL_TAIL)], btail)
        pltpu.sync_copy(xf_h.at[pl.ds(coff + base, _PL_TAIL)],
                        stage.at[pl.ds(0, _PL_TAIL)])
        pltpu.sync_copy(stage.at[pl.ds(0, _PL_TAIL)],
                        sumS.at[btail], add=True)

        @pl.when(ci == 0)
        def _cntt():
            pltpu.sync_copy(onesv.at[pl.ds(0, _PL_TAIL)],
                            cntS.at[btail], add=True)

    plsc.subcore_barrier()

    @pl.when(s == 0)
    def _out():
        pltpu.sync_copy(sumS, zbv)
        pltpu.sync_copy(zbv, sums_h.at[ci])

    @pl.when(jnp.logical_and(s == 0, ci == 0))
    def _outc():
        pltpu.sync_copy(cntS, zcv)
        pltpu.sync_copy(zcv, cnt_h)


def _sc_pool(xflat, batch):
    k = pl.kernel(
        _pool_body,
        out_type=[
            jax.ShapeDtypeStruct((2, _B, _HF), _f32),
            jax.ShapeDtypeStruct((_B, 16), _f32),
        ],
        mesh=_mesh(),
        compiler_params=pltpu.CompilerParams(use_tc_tiling_on_sc=False),
        scratch_types=[
            pltpu.VMEM((128,), _i32),
            pltpu.VMEM((_PL_TAIL,), _i32),
            pltpu.VMEM((128, _HF), _f32),
            pltpu.VMEM((128, 16), _f32),
            pltpu.VMEM((_B, _HF), _f32),
            pltpu.VMEM((_B, 16), _f32),
            pltpu.VMEM_SHARED((_B, _HF), _f32),
            pltpu.VMEM_SHARED((_B, 16), _f32),
        ],
    )
    return k(xflat, batch)


def kernel(x, edge_index, edge_weight, batch, W1, b1, W2, b2, We, be, Wf, bf, lew):
    del edge_weight
    row = edge_index[0]
    col = edge_index[1]

    lew_pad = jnp.pad(lew, ((0, _RP - _R), (0, _RP - _R)))
    tflat = _tc_table(lew_pad).reshape(-1)

    w, degpf = _sc_wdeg(row, col, tflat)
    dinv8 = _tc_deg(degpf.reshape(16, _BN))
    dvcol = dinv8.reshape(-1, 1)

    hs1, xe = _tc_mm(x, W1, We, be.reshape(1, _D), dvcol)
    agg1 = _sc_agg(row, col, w, hs1.reshape(2 * _N, _HF))
    x1, hs2 = _tc_l1(agg1.reshape(2, _N, _HF), hs1, xe, dvcol,
                     b1.reshape(1, _D), W2)
    agg2 = _sc_agg(row, col, w, hs2.reshape(2 * _N, _HF))
    x2 = _tc_l2(agg2.reshape(2, _N, _HF), hs2, x1, dvcol, b2.reshape(1, _D))

    sums, cnt16 = _sc_pool(x2.reshape(2 * _N, _HF), batch)
    return _tc_pred(sums, cnt16, Wf, bf.reshape(1, 1))


# speedup vs baseline: 1.8740x; 1.0889x over previous
import functools

import jax
import jax.numpy as jnp
from jax import lax
from jax.experimental import pallas as pl
from jax.experimental.pallas import tpu as pltpu
from jax.experimental.pallas import tpu_sc as plsc

_N = 24256
_E = 388096
_D = 128
_B = 64
_R = 379
_RP = 384
_HF = 64
_NT = 16
_NC = 2
_BN = 3032

_f32 = jnp.float32
_i32 = jnp.int32


def _mesh():
    return plsc.VectorSubcoreMesh(core_axis_name="c", subcore_axis_name="s")


def _table_body(lew_ref, t_ref):
    a = lew_ref[...]
    t_ref[...] = 2.0 * jax.nn.sigmoid((a + a.T) * 0.5)


def _tc_table(lew_pad):
    return pl.pallas_call(
        _table_body,
        out_shape=jax.ShapeDtypeStruct((_RP, _RP), _f32),
    )(lew_pad)


def _dv_from(dp):
    return lax.rsqrt(dp[0] + dp[1] + 1.0)


def _mm_body(x_ref, w1_ref, we_ref, be_ref, dp_ref, hs1_ref, xe_ref):
    xb = x_ref[...]
    dv = _dv_from(dp_ref[...])
    hs1 = jnp.dot(xb, w1_ref[...], preferred_element_type=_f32) * dv
    xe = jnp.dot(xb, we_ref[...], preferred_element_type=_f32) + be_ref[...]
    xe = jnp.maximum(xe, 0.0)
    hs1_ref[0] = hs1[:, :_HF]
    hs1_ref[1] = hs1[:, _HF:]
    xe_ref[0] = xe[:, :_HF]
    xe_ref[1] = xe[:, _HF:]


def _tc_mm(x, w1, we, be_row, degp2):
    grid = _N // _BN
    return pl.pallas_call(
        _mm_body,
        grid=(grid,),
        in_specs=[
            pl.BlockSpec((_BN, _D), lambda i: (i, 0)),
            pl.BlockSpec((_D, _D), lambda i: (0, 0)),
            pl.BlockSpec((_D, _D), lambda i: (0, 0)),
            pl.BlockSpec((1, _D), lambda i: (0, 0)),
            pl.BlockSpec((2, _BN, 1), lambda i: (0, i, 0)),
        ],
        out_specs=[
            pl.BlockSpec((2, _BN, _HF), lambda i: (0, i, 0)),
            pl.BlockSpec((2, _BN, _HF), lambda i: (0, i, 0)),
        ],
        out_shape=[
            jax.ShapeDtypeStruct((2, _N, _HF), _f32),
            jax.ShapeDtypeStruct((2, _N, _HF), _f32),
        ],
    )(x, w1, we, be_row, degp2)


def _l1_body(agg_ref, hs1_ref, xe_ref, dp_ref, b1_ref, w2_ref, x1_ref, hs2_ref):
    dv = _dv_from(dp_ref[...])
    b1 = b1_ref[...]
    x1h = []
    for t in range(2):
        o = (agg_ref[t] + hs1_ref[t]) * dv + b1[:, _HF * t:_HF * (t + 1)]
        x1h.append(jnp.maximum(o, 0.0) + xe_ref[t])
    w2 = w2_ref[...]
    hs2 = (jnp.dot(x1h[0], w2[:_HF, :], preferred_element_type=_f32)
           + jnp.dot(x1h[1], w2[_HF:, :], preferred_element_type=_f32)) * dv
    x1_ref[0] = x1h[0]
    x1_ref[1] = x1h[1]
    hs2_ref[0] = hs2[:, :_HF]
    hs2_ref[1] = hs2[:, _HF:]


def _tc_l1(agg1, h1, xe, d2col, b1_row, w2):
    grid = _N // _BN
    half_spec = pl.BlockSpec((2, _BN, _HF), lambda i: (0, i, 0))
    return pl.pallas_call(
        _l1_body,
        grid=(grid,),
        in_specs=[
            half_spec,
            half_spec,
            half_spec,
            pl.BlockSpec((2, _BN, 1), lambda i: (0, i, 0)),
            pl.BlockSpec((1, _D), lambda i: (0, 0)),
            pl.BlockSpec((_D, _D), lambda i: (0, 0)),
        ],
        out_specs=[half_spec, half_spec],
        out_shape=[
            jax.ShapeDtypeStruct((2, _N, _HF), _f32),
            jax.ShapeDtypeStruct((2, _N, _HF), _f32),
        ],
    )(agg1, h1, xe, d2col, b1_row, w2)


def _l2_body(agg_ref, hs2_ref, x1_ref, dp_ref, b2_ref, x2_ref):
    dv = _dv_from(dp_ref[...])
    b2 = b2_ref[...]
    for t in range(2):
        o = (agg_ref[t] + hs2_ref[t]) * dv + b2[:, _HF * t:_HF * (t + 1)]
        x2_ref[t] = jnp.maximum(o, 0.0) + x1_ref[t]


def _tc_l2(agg2, h2, x1, d2col, b2_row):
    grid = _N // _BN
    half_spec = pl.BlockSpec((2, _BN, _HF), lambda i: (0, i, 0))
    return pl.pallas_call(
        _l2_body,
        grid=(grid,),
        in_specs=[
            half_spec,
            half_spec,
            half_spec,
            pl.BlockSpec((2, _BN, 1), lambda i: (0, i, 0)),
            pl.BlockSpec((1, _D), lambda i: (0, 0)),
        ],
        out_specs=half_spec,
        out_shape=jax.ShapeDtypeStruct((2, _N, _HF), _f32),
    )(agg2, h2, x1, d2col, b2_row)


def _pred_body(sums_ref, cnt_ref, wf_ref, bf_ref, out_ref):
    cnt = jnp.maximum(cnt_ref[...][:, 0:1], 1.0)
    wf = wf_ref[...]
    p0 = sums_ref[0] / cnt
    p1 = sums_ref[1] / cnt
    out_ref[...] = (jnp.dot(p0, wf[:_HF, :], preferred_element_type=_f32)
                    + jnp.dot(p1, wf[_HF:, :], preferred_element_type=_f32)
                    + bf_ref[...])


def _tc_pred(sums, cnt16, wf, bf_row):
    return pl.pallas_call(
        _pred_body,
        out_shape=jax.ShapeDtypeStruct((_B, 1), _f32),
    )(sums, cnt16, wf, bf_row)


_EW_TILE = _E // (_NC * _NT)
_EW_CH = 128
_EW_FULL = _EW_TILE // _EW_CH
_EW_TAIL = _EW_TILE - _EW_FULL * _EW_CH

_DEG_SL = _N // 8


_EW_PIPE = _EW_FULL - 1
_EW_K = (_EW_PIPE - 1) // 2


def _wdeg_body(row_h, col_h, tflat_h, w_h, degp_h,
               rbuf, cbuf, fbuf, wstage,
               rbuf2, cbuf2, fbuf2, wstage2,
               rtb, ctb, ftb, wtb,
               semi0, semi1, semt0, semt1, zb, degS):
    ci = lax.axis_index("c")
    s = lax.axis_index("s")
    wid = ci * _NT + s
    tbase = wid * _EW_TILE

    @pl.loop(0, 192)
    def _z(i):
        zb[pl.ds(i * 16, 16)] = jnp.zeros((16,), _f32)

    @pl.when(s < 8)
    def _zdeg():
        pltpu.sync_copy(zb.at[pl.ds(0, _DEG_SL)],
                        degS.at[pl.ds(s * _DEG_SL, _DEG_SL)])

    plsc.subcore_barrier()

    def _fcompute(ch, rb, cb, fb):
        @pl.loop(0, ch // 16)
        def _fg(g):
            sl = pl.ds(g * 16, 16)
            fb[sl] = (rb[sl] % _R) * _RP + (cb[sl] % _R)

    def _chunk_sync(base, ch, rb, cb, fb, wst):
        pltpu.sync_copy(row_h.at[pl.ds(base, ch)], rb)
        pltpu.sync_copy(col_h.at[pl.ds(base, ch)], cb)
        _fcompute(ch, rb, cb, fb)
        pltpu.sync_copy(tflat_h.at[fb], wst)
        pltpu.sync_copy(wst, w_h.at[pl.ds(base, ch)])
        pltpu.sync_copy(wst, degS.at[cb], add=True)

    _chunk_sync(tbase, _EW_TAIL, rtb, ctb, ftb, wtb)
    _chunk_sync(tbase + _EW_TAIL, _EW_CH, rbuf, cbuf, fbuf, wstage)

    pbase = tbase + _EW_TAIL + _EW_CH
    rb = (rbuf, rbuf2)
    cb = (cbuf, cbuf2)
    fb = (fbuf, fbuf2)
    wst = (wstage, wstage2)
    semi = (semi0, semi1)
    semt = (semt0, semt1)

    def start_idx(i, bs):
        base = pbase + i * _EW_CH
        pltpu.async_copy(row_h.at[pl.ds(base, _EW_CH)], rb[bs], semi[bs])
        pltpu.async_copy(col_h.at[pl.ds(base, _EW_CH)], cb[bs], semi[bs])

    def wait_idx(bs):
        pltpu.make_async_copy(row_h.at[pl.ds(tbase, _EW_CH)], rb[bs],
                              semi[bs]).wait()
        pltpu.make_async_copy(col_h.at[pl.ds(tbase, _EW_CH)], cb[bs],
                              semi[bs]).wait()

    def tg_start(bs):
        _fcompute(_EW_CH, rb[bs], cb[bs], fb[bs])
        pltpu.async_copy(tflat_h.at[fb[bs]], wst[bs], semt[bs])

    def tg_wait(bs):
        pltpu.make_async_copy(tflat_h.at[fb[bs]], wst[bs], semt[bs]).wait()

    def emit(i, bs):
        base = pbase + i * _EW_CH
        pltpu.sync_copy(wst[bs], w_h.at[pl.ds(base, _EW_CH)])
        pltpu.sync_copy(wst[bs], degS.at[cb[bs]], add=True)

    start_idx(0, 0)
    wait_idx(0)
    tg_start(0)
    start_idx(1, 1)

    @pl.loop(0, _EW_K)
    def _pair(k):
        i2 = k * 2
        wait_idx(1)
        tg_start(1)
        tg_wait(0)
        emit(i2, 0)
        start_idx(i2 + 2, 0)
        wait_idx(0)
        tg_start(0)
        tg_wait(1)
        emit(i2 + 1, 1)

        @pl.when(k < _EW_K - 1)
        def _pf():
            start_idx(i2 + 3, 1)

    tg_wait(0)
    emit(_EW_PIPE - 1, 0)

    plsc.subcore_barrier()

    @pl.when(s < 8)
    def _out():
        pltpu.sync_copy(degS.at[pl.ds(s * _DEG_SL, _DEG_SL)],
                        zb.at[pl.ds(0, _DEG_SL)])
        pltpu.sync_copy(zb.at[pl.ds(0, _DEG_SL)],
                        degp_h.at[pl.ds(ci * _N + s * _DEG_SL, _DEG_SL)])


def _sc_wdeg(row, col, tflat):
    k = pl.kernel(
        _wdeg_body,
        out_type=[
            jax.ShapeDtypeStruct((_E,), _f32),
            jax.ShapeDtypeStruct((2 * _N,), _f32),
        ],
        mesh=_mesh(),
        compiler_params=pltpu.CompilerParams(use_tc_tiling_on_sc=False),
        scratch_types=[
            pltpu.VMEM((_EW_CH,), _i32),
            pltpu.VMEM((_EW_CH,), _i32),
            pltpu.VMEM((_EW_CH,), _i32),
            pltpu.VMEM((_EW_CH,), _f32),
            pltpu.VMEM((_EW_CH,), _i32),
            pltpu.VMEM((_EW_CH,), _i32),
            pltpu.VMEM((_EW_CH,), _i32),
            pltpu.VMEM((_EW_CH,), _f32),
            pltpu.VMEM((_EW_TAIL,), _i32),
            pltpu.VMEM((_EW_TAIL,), _i32),
            pltpu.VMEM((_EW_TAIL,), _i32),
            pltpu.VMEM((_EW_TAIL,), _f32),
            pltpu.SemaphoreType.DMA,
            pltpu.SemaphoreType.DMA,
            pltpu.SemaphoreType.DMA,
            pltpu.SemaphoreType.DMA,
            pltpu.VMEM((3072,), _f32),
            pltpu.VMEM_SHARED((_N,), _f32),
        ],
    )
    return k(row, col, tflat)


_AG_TILE = _E // _NT
_AG_CH = 128
_AG_TAIL = 64
_AG_FULL = (_AG_TILE - _AG_TAIL) // _AG_CH
_AG_K = (_AG_FULL - 1) // 2
_ROWS_T = 1520
_ROWS_LAST = _N - 15 * _ROWS_T


def _agg_body(row_h, col_h, w_h, hf_h, agg_h,
              rbuf, cbuf, gbuf, wbuf, stage,
              rbuf2, cbuf2, gbuf2, wbuf2, stage2,
              rT, cT, gT, wT, stageT, csb, csb2,
              semi0, semi1, semg0, semg1, sems0, sems1, zbuf, acc):
    ci = lax.axis_index("c")
    s = lax.axis_index("s")
    coff = ci * _N
    tbase = s * _AG_TILE

    @pl.loop(0, 128)
    def _z(i):
        for q in range(4):
            zbuf[i, pl.ds(q * 16, 16)] = jnp.zeros((16,), _f32)

    rbase = pl.multiple_of(s * _ROWS_T, 8)
    for kk in range(11):
        pltpu.sync_copy(zbuf, acc.at[pl.ds(rbase + kk * 128, 128)])

    @pl.when(s < _NT - 1)
    def _ztail():
        pltpu.sync_copy(zbuf.at[pl.ds(0, 112)],
                        acc.at[pl.ds(rbase + 1408, 112)])

    @pl.when(s == _NT - 1)
    def _ztail_last():
        pltpu.sync_copy(zbuf.at[pl.ds(0, 48)],
                        acc.at[pl.ds(rbase + 1408, 48)])

    plsc.subcore_barrier()

    rb = (rbuf, rbuf2)
    cb = (cbuf, cbuf2)
    gb = (gbuf, gbuf2)
    wb = (wbuf, wbuf2)
    st = (stage, stage2)
    semi = (semi0, semi1)
    semg = (semg0, semg1)
    fbase = tbase + _AG_TAIL

    def scale(stg, wref, n):
        for g4 in range(n // 16):
            wv = wref[pl.ds(g4 * 16, 16)]
            for l in range(16):
                e = g4 * 16 + l
                nb = wv.at[lax.full((16,), l, _i32)].get(
                    mode="promise_in_bounds")
                for q in range(4):
                    sl2 = pl.ds(q * 16, 16)
                    stg[e, sl2] = stg[e, sl2] * nb

    pltpu.sync_copy(row_h.at[pl.ds(tbase, _AG_TAIL)], rT)
    pltpu.sync_copy(col_h.at[pl.ds(tbase, _AG_TAIL)], cT)
    pltpu.sync_copy(w_h.at[pl.ds(tbase, _AG_TAIL)], wT)
    for g in range(_AG_TAIL // 16):
        sl = pl.ds(g * 16, 16)
        gT[sl] = rT[sl] + coff
    pltpu.sync_copy(hf_h.at[gT], stageT)
    scale(stageT, wT, _AG_TAIL)
    pltpu.sync_copy(stageT, acc.at[cT], add=True)

    def start_idx(i, bs):
        base = fbase + i * _AG_CH
        pltpu.async_copy(row_h.at[pl.ds(base, _AG_CH)], rb[bs], semi[bs])
        pltpu.async_copy(col_h.at[pl.ds(base, _AG_CH)], cb[bs], semi[bs])
        pltpu.async_copy(w_h.at[pl.ds(base, _AG_CH)], wb[bs], semi[bs])

    def wait_idx(bs):
        pltpu.make_async_copy(row_h.at[pl.ds(tbase, _AG_CH)], rb[bs],
                              semi[bs]).wait()
        pltpu.make_async_copy(col_h.at[pl.ds(tbase, _AG_CH)], cb[bs],
                              semi[bs]).wait()
        pltpu.make_async_copy(w_h.at[pl.ds(tbase, _AG_CH)], wb[bs],
                              semi[bs]).wait()

    def gather_start(bs):
        for g in range(_AG_CH // 16):
            sl = pl.ds(g * 16, 16)
            gb[bs][sl] = rb[bs][sl] + coff
        pltpu.async_copy(hf_h.at[gb[bs]], st[bs], semg[bs])

    def gather_wait(bs):
        pltpu.make_async_copy(hf_h.at[gb[bs]], st[bs], semg[bs]).wait()

    cs = (csb, csb2)
    sems = (sems0, sems1)

    def scale_scatter(bs):
        scale(st[bs], wb[bs], _AG_CH)
        for g in range(_AG_CH // 16):
            sl = pl.ds(g * 16, 16)
            cs[bs][sl] = cb[bs][sl]
        pltpu.async_copy(st[bs], acc.at[cs[bs]], sems[bs], add=True)

    def scat_drain(bs):
        pltpu.make_async_copy(st[bs], acc.at[cs[bs]], sems[bs]).wait()

    start_idx(0, 0)
    wait_idx(0)
    gather_start(0)
    start_idx(1, 1)

    @pl.loop(0, _AG_K)
    def _pair(k):
        i2 = k * 2
        wait_idx(1)

        @pl.when(k > 0)
        def _dr1():
            scat_drain(1)

        gather_start(1)
        gather_wait(0)
        scale_scatter(0)
        start_idx(i2 + 2, 0)
        wait_idx(0)
        scat_drain(0)
        gather_start(0)
        gather_wait(1)
        scale_scatter(1)

        @pl.when(k < _AG_K - 1)
        def _pf():
            start_idx(i2 + 3, 1)

    gather_wait(0)
    scale_scatter(0)
    scat_drain(0)
    scat_drain(1)

    plsc.subcore_barrier()

    obase = pl.multiple_of(coff + rbase, 8)
    for kk in range(11):
        pltpu.sync_copy(acc.at[pl.ds(rbase + kk * 128, 128)], zbuf)
        pltpu.sync_copy(zbuf, agg_h.at[pl.ds(obase + kk * 128, 128)])

    @pl.when(s < _NT - 1)
    def _otail():
        pltpu.sync_copy(acc.at[pl.ds(rbase + 1408, 112)],
                        zbuf.at[pl.ds(0, 112)])
        pltpu.sync_copy(zbuf.at[pl.ds(0, 112)],
                        agg_h.at[pl.ds(obase + 1408, 112)])

    @pl.when(s == _NT - 1)
    def _otail_last():
        pltpu.sync_copy(acc.at[pl.ds(rbase + 1408, 48)],
                        zbuf.at[pl.ds(0, 48)])
        pltpu.sync_copy(zbuf.at[pl.ds(0, 48)],
                        agg_h.at[pl.ds(obase + 1408, 48)])


def _sc_agg(row, col, w, hflat):
    k = pl.kernel(
        _agg_body,
        out_type=jax.ShapeDtypeStruct((2 * _N, _HF), _f32),
        mesh=_mesh(),
        compiler_params=pltpu.CompilerParams(use_tc_tiling_on_sc=False),
        scratch_types=[
            pltpu.VMEM((_AG_CH,), _i32),
            pltpu.VMEM((_AG_CH,), _i32),
            pltpu.VMEM((_AG_CH,), _i32),
            pltpu.VMEM((_AG_CH,), _f32),
            pltpu.VMEM((_AG_CH, _HF), _f32),
            pltpu.VMEM((_AG_CH,), _i32),
            pltpu.VMEM((_AG_CH,), _i32),
            pltpu.VMEM((_AG_CH,), _i32),
            pltpu.VMEM((_AG_CH,), _f32),
            pltpu.VMEM((_AG_CH, _HF), _f32),
            pltpu.VMEM((_AG_TAIL,), _i32),
            pltpu.VMEM((_AG_TAIL,), _i32),
            pltpu.VMEM((_AG_TAIL,), _i32),
            pltpu.VMEM((_AG_TAIL,), _f32),
            pltpu.VMEM((_AG_TAIL, _HF), _f32),
            pltpu.VMEM((_AG_CH,), _i32),
            pltpu.VMEM((_AG_CH,), _i32),
            pltpu.SemaphoreType.DMA,
            pltpu.SemaphoreType.DMA,
            pltpu.SemaphoreType.DMA,
            pltpu.SemaphoreType.DMA,
            pltpu.SemaphoreType.DMA,
            pltpu.SemaphoreType.DMA,
            pltpu.VMEM((128, _HF), _f32),
            pltpu.VMEM_SHARED((_N, _HF), _f32),
        ],
    )
    return k(row, col, w, hflat)


_PL_FULL = _N // 128
_PL_TAIL = _N - _PL_FULL * 128


def _pool_body(xf_h, batch_h, sums_h, cnt_h,
               bbuf, btail, stage, onesv, zbv, zcv, sumS, cntS):
    ci = lax.axis_index("c")
    s = lax.axis_index("s")
    coff = ci * _N

    @pl.loop(0, 64)
    def _z(i):
        for q in range(4):
            zbv[i, pl.ds(q * 16, 16)] = jnp.zeros((16,), _f32)

    @pl.loop(0, 128)
    def _o(i):
        onesv[i, pl.ds(0, 16)] = jnp.ones((16,), _f32)

    @pl.loop(0, 64)
    def _zc(i):
        zcv[i, pl.ds(0, 16)] = jnp.zeros((16,), _f32)

    @pl.when(s == 0)
    def _zero():
        pltpu.sync_copy(zbv, sumS)

    @pl.when(jnp.logical_and(s == 0, ci == 0))
    def _zeroc():
        pltpu.sync_copy(zcv, cntS)

    plsc.subcore_barrier()

    nch = (204 - s) // 16

    @pl.loop(0, nch)
    def _c(k):
        j = s + k * _NT
        base = j * 128
        pltpu.sync_copy(batch_h.at[pl.ds(base, 128)], bbuf)
        pltpu.sync_copy(xf_h.at[pl.ds(coff + base, 128)], stage)
        pltpu.sync_copy(stage, sumS.at[bbuf], add=True)

        @pl.when(ci == 0)
        def _cnt():
            pltpu.sync_copy(onesv, cntS.at[bbuf], add=True)

    @pl.when(s == _NT - 1)
    def _tail():
        base = _PL_FULL * 128
        pltpu.sync_copy(batch_h.at[pl.ds(base, _PL_TAIL)], btail)
        pltpu.sync_copy(xf_h.at[pl.ds(coff + base, _PL_TAIL)],
                        stage.at[pl.ds(0, _PL_TAIL)])
        pltpu.sync_copy(stage.at[pl.ds(0, _PL_TAIL)],
                        sumS.at[btail], add=True)

        @pl.when(ci == 0)
        def _cntt():
            pltpu.sync_copy(onesv.at[pl.ds(0, _PL_TAIL)],
                            cntS.at[btail], add=True)

    plsc.subcore_barrier()

    @pl.when(s == 0)
    def _out():
        pltpu.sync_copy(sumS, zbv)
        pltpu.sync_copy(zbv, sums_h.at[ci])

    @pl.when(jnp.logical_and(s == 0, ci == 0))
    def _outc():
        pltpu.sync_copy(cntS, zcv)
        pltpu.sync_copy(zcv, cnt_h)


def _sc_pool(xflat, batch):
    k = pl.kernel(
        _pool_body,
        out_type=[
            jax.ShapeDtypeStruct((2, _B, _HF), _f32),
            jax.ShapeDtypeStruct((_B, 16), _f32),
        ],
        mesh=_mesh(),
        compiler_params=pltpu.CompilerParams(use_tc_tiling_on_sc=False),
        scratch_types=[
            pltpu.VMEM((128,), _i32),
            pltpu.VMEM((_PL_TAIL,), _i32),
            pltpu.VMEM((128, _HF), _f32),
            pltpu.VMEM((128, 16), _f32),
            pltpu.VMEM((_B, _HF), _f32),
            pltpu.VMEM((_B, 16), _f32),
            pltpu.VMEM_SHARED((_B, _HF), _f32),
            pltpu.VMEM_SHARED((_B, 16), _f32),
        ],
    )
    return k(xflat, batch)


def kernel(x, edge_index, edge_weight, batch, W1, b1, W2, b2, We, be, Wf, bf, lew):
    del edge_weight
    row = edge_index[0]
    col = edge_index[1]

    lew_pad = jnp.pad(lew, ((0, _RP - _R), (0, _RP - _R)))
    tflat = _tc_table(lew_pad).reshape(-1)

    w, degpf = _sc_wdeg(row, col, tflat)
    degp2 = degpf.reshape(2, _N, 1)

    hs1, xe = _tc_mm(x, W1, We, be.reshape(1, _D), degp2)
    agg1 = _sc_agg(row, col, w, hs1.reshape(2 * _N, _HF))
    x1, hs2 = _tc_l1(agg1.reshape(2, _N, _HF), hs1, xe, degp2,
                     b1.reshape(1, _D), W2)
    agg2 = _sc_agg(row, col, w, hs2.reshape(2 * _N, _HF))
    x2 = _tc_l2(agg2.reshape(2, _N, _HF), hs2, x1, degp2, b2.reshape(1, _D))

    sums, cnt16 = _sc_pool(x2.reshape(2 * _N, _HF), batch)
    return _tc_pred(sums, cnt16, Wf, bf.reshape(1, 1))
